# baseline (speedup 1.0000x reference)
import jax
import jax.numpy as jnp
from jax import lax
from jax.experimental import pallas as pl
from jax.experimental.pallas import tpu as pltpu
from jax.experimental.pallas import tpu_sc as plsc

VOCAB = 1000000
D = 64
B = 4096
S = 200
SA = 128
SB = S - SA
NC = 2
NS = 16
NW = NC * NS
ROWS_PER_W = B // NW


def _body(table_hbm, pos_hbm, idxa_hbm, idxb_hbm, out_hbm,
          idxa_v, idxb_v, out_buf, pos_sh, gsem):
    cid = lax.axis_index("c")
    sid = lax.axis_index("s")
    wid = sid * NC + cid
    base = wid * ROWS_PER_W

    @pl.when(sid == 0)
    def _():
        pltpu.sync_copy(pos_hbm.at[pl.ds(0, S)], pos_sh)

    pltpu.sync_copy(idxa_hbm.at[pl.ds(base, ROWS_PER_W)], idxa_v)
    pltpu.sync_copy(idxb_hbm.at[pl.ds(base, ROWS_PER_W)], idxb_v)
    plsc.subcore_barrier()

    def step(i, carry):
        pltpu.sync_copy(pos_sh, out_buf)
        cp_a = pltpu.async_copy(
            table_hbm.at[idxa_v.at[i]], out_buf.at[pl.ds(0, SA)], gsem,
            add=True)
        cp_b = pltpu.async_copy(
            table_hbm.at[idxb_v.at[i]], out_buf.at[pl.ds(SA, SB)], gsem,
            add=True)
        cp_a.wait()
        cp_b.wait()
        pltpu.sync_copy(out_buf, out_hbm.at[pl.ds((base + i) * S, S)])
        return carry

    lax.fori_loop(0, ROWS_PER_W, step, 0)


def kernel(x, embedding_weight, positional_weight):
    idxa = x[:, :SA].astype(jnp.int32)
    idxb = x[:, SA:].astype(jnp.int32)
    mesh = plsc.VectorSubcoreMesh(core_axis_name="c", subcore_axis_name="s")
    out = pl.kernel(
        _body,
        out_type=jax.ShapeDtypeStruct((B * S, D), jnp.float32),
        mesh=mesh,
        scratch_types=[
            pltpu.VMEM((ROWS_PER_W, SA), jnp.int32),
            pltpu.VMEM((ROWS_PER_W, SB), jnp.int32),
            pltpu.VMEM((S, D), jnp.float32),
            pltpu.MemorySpace.VMEM_SHARED((S, D), jnp.float32),
            pltpu.SemaphoreType.DMA,
        ],
        compiler_params=pltpu.CompilerParams(use_tc_tiling_on_sc=False),
    )(embedding_weight, positional_weight, idxa, idxb)
    return out.reshape(B, S, D)


# speedup vs baseline: 1.1255x; 1.1255x over previous
import jax
import jax.numpy as jnp
from jax import lax
from jax.experimental import pallas as pl
from jax.experimental.pallas import tpu as pltpu
from jax.experimental.pallas import tpu_sc as plsc

VOCAB = 1000000
D = 64
B = 4096
S = 200
SA = 128
SB = S - SA
NC = 2
NS = 16
NW = NC * NS
ROWS_PER_W = B // NW
G = 2
NCHUNK = ROWS_PER_W // G
NBUF = 4


def _body(table_hbm, pos_hbm, idxa_hbm, idxb_hbm, out_hbm,
          idxa_v, idxb_v, buf0, buf1, buf2, buf3, pos_sh,
          g0, g1, g2, g3, o0, o1, o2, o3):
    bufs = (buf0, buf1, buf2, buf3)
    gsems = (g0, g1, g2, g3)
    osems = (o0, o1, o2, o3)

    cid = lax.axis_index("c")
    sid = lax.axis_index("s")
    wid = sid * NC + cid
    base = wid * ROWS_PER_W

    @pl.when(sid == 0)
    def _():
        for r in range(G):
            pltpu.sync_copy(pos_hbm.at[pl.ds(0, S)],
                            pos_sh.at[pl.ds(r * S, S)])

    pltpu.sync_copy(idxa_hbm.at[pl.ds(base, ROWS_PER_W)], idxa_v)
    pltpu.sync_copy(idxb_hbm.at[pl.ds(base, ROWS_PER_W)], idxb_v)
    plsc.subcore_barrier()

    def gathers(g, s):
        row = G * g
        cps = []
        for r in range(G):
            cps.append(pltpu.make_async_copy(
                table_hbm.at[idxa_v.at[row + r]],
                bufs[s].at[pl.ds(r * S, SA)], gsems[s]))
            cps.append(pltpu.make_async_copy(
                table_hbm.at[idxb_v.at[row + r]],
                bufs[s].at[pl.ds(r * S + SA, SB)], gsems[s]))
        return cps

    def store(g, s):
        return pltpu.make_async_copy(
            bufs[s], out_hbm.at[pl.ds((base + G * g) * S, G * S)], osems[s])

    def step(gg, carry):
        for s in range(NBUF):
            g = gg * NBUF + s
            sp = (s + NBUF - 1) % NBUF

            @pl.when(g >= NBUF)
            def _():
                store(jnp.maximum(g - NBUF, 0), s).wait()

            pltpu.sync_copy(pos_sh, bufs[s])
            for cp in gathers(g, s):
                cp.start(add=True)

            @pl.when(g >= 1)
            def _():
                gp = jnp.maximum(g - 1, 0)
                for cp in gathers(gp, sp):
                    cp.wait()
                store(gp, sp).start()
        return carry

    lax.fori_loop(0, NCHUNK // NBUF, step, 0)

    last = NCHUNK - 1
    for cp in gathers(last, (NCHUNK - 1) % NBUF):
        cp.wait()
    store(last, (NCHUNK - 1) % NBUF).start()
    for s in range(NBUF):
        store(NCHUNK - NBUF + s, s).wait()


def kernel(x, embedding_weight, positional_weight):
    idxa = x[:, :SA].astype(jnp.int32)
    idxb = x[:, SA:].astype(jnp.int32)
    mesh = plsc.VectorSubcoreMesh(core_axis_name="c", subcore_axis_name="s")
    out = pl.kernel(
        _body,
        out_type=jax.ShapeDtypeStruct((B * S, D), jnp.float32),
        mesh=mesh,
        scratch_types=[
            pltpu.VMEM((ROWS_PER_W, SA), jnp.int32),
            pltpu.VMEM((ROWS_PER_W, SB), jnp.int32),
        ] + [pltpu.VMEM((G * S, D), jnp.float32) for _ in range(NBUF)] + [
            pltpu.MemorySpace.VMEM_SHARED((G * S, D), jnp.float32),
        ] + [pltpu.SemaphoreType.DMA for _ in range(2 * NBUF)],
        compiler_params=pltpu.CompilerParams(use_tc_tiling_on_sc=False),
    )(embedding_weight, positional_weight, idxa, idxb)
    return out.reshape(B, S, D)
